# baseline (speedup 1.0000x reference)
import functools

import jax
import jax.numpy as jnp
from jax import lax
from jax.experimental import pallas as pl
from jax.experimental.pallas import tpu as pltpu
from jax.experimental.pallas import tpu_sc as plsc

N, E, D, C = 10000, 320000, 128, 64
H = 4 * C

NC, NS, L = 2, 16, 16
NW = NC * NS
NPAD = 10240
RPT = NPAD // NS
KB = 64
EPW = 10112
NBATCH = EPW // KB
KB2 = 128
NB2 = EPW // KB2
LAST_RPT = N - (NS - 1) * RPT

NBT4 = 2 * NBATCH
NB4_0, NB4_1 = 140, 176
NBMAX4 = max(NB4_0, NB4_1)
NBT2 = 2 * NB2
NB2_0, NB2_1 = 56, 102
NBMAX2 = max(NB2_0, NB2_1)
assert NB4_0 + NB4_1 == NBT4 and NB4_0 % 2 == 0 and NB4_1 % 2 == 0
assert NB2_0 + NB2_1 == NBT2 and NB2_0 % 2 == 0 and NB2_1 % 2 == 0
EPAD = EPW * NW

_mesh = plsc.VectorSubcoreMesh(
    core_axis_name="c", subcore_axis_name="s", num_cores=NC, num_subcores=NS)


def _tc1_body(x_ref, w_ref, o_ref):
    o_ref[...] = jnp.dot(x_ref[...], w_ref[...],
                         preferred_element_type=jnp.float32,
                         precision=jax.lax.Precision.HIGHEST)


def _tc1_y(x, w_gcn):
    rb = 1000
    return pl.pallas_call(
        _tc1_body,
        grid=(N // rb,),
        in_specs=[
            pl.BlockSpec((rb, D), lambda i: (i, 0)),
            pl.BlockSpec((D, C), lambda i: (0, 0)),
        ],
        out_specs=pl.BlockSpec((rb, C), lambda i: (i, 0)),
        out_shape=jax.ShapeDtypeStruct((N, C), jnp.float32),
    )(x, w_gcn)


def _tc3_body(p_ref, bg_ref, w1a_ref, w1b_ref, b1_ref, u_ref, v_ref):
    emb = p_ref[0] + p_ref[1] + bg_ref[...]
    u_ref[...] = jnp.dot(emb, w1a_ref[...],
                         preferred_element_type=jnp.float32,
                         precision=jax.lax.Precision.HIGHEST) + b1_ref[...]
    v_ref[...] = jnp.dot(emb, w1b_ref[...],
                         preferred_element_type=jnp.float32,
                         precision=jax.lax.Precision.HIGHEST)


def _tc3_uv(partial, b_gcn, w1a, w1b, b1):
    rb = 1000
    return pl.pallas_call(
        _tc3_body,
        grid=(N // rb,),
        in_specs=[
            pl.BlockSpec((NC, rb, C), lambda i: (0, i, 0)),
            pl.BlockSpec((1, C), lambda i: (0, 0)),
            pl.BlockSpec((C, H), lambda i: (0, 0)),
            pl.BlockSpec((C, H), lambda i: (0, 0)),
            pl.BlockSpec((1, H), lambda i: (0, 0)),
        ],
        out_specs=[
            pl.BlockSpec((rb, H), lambda i: (i, 0)),
            pl.BlockSpec((rb, H), lambda i: (i, 0)),
        ],
        out_shape=[
            jax.ShapeDtypeStruct((N, H), jnp.float32),
            jax.ShapeDtypeStruct((N, H), jnp.float32),
        ],
    )(partial, b_gcn, w1a, w1b, b1)


def _tc5_body(q_ref, bg_ref, o_ref):
    o_ref[...] = q_ref[0] + q_ref[1] + bg_ref[...]


def _tc5_out(partial, b_gcn):
    rb = 1000
    return pl.pallas_call(
        _tc5_body,
        grid=(N // rb,),
        in_specs=[
            pl.BlockSpec((NC, rb, C), lambda i: (0, i, 0)),
            pl.BlockSpec((1, C), lambda i: (0, 0)),
        ],
        out_specs=pl.BlockSpec((rb, C), lambda i: (i, 0)),
        out_shape=jax.ShapeDtypeStruct((N, C), jnp.float32),
    )(partial, b_gcn)


_SHIFT = 14
_MASK = (1 << _SHIFT) - 1


def _decode_batch(comb_v, j, src_ref, dst_ref, kb=KB):
    for c in range(kb // L):
        sl = pl.ds(c * L, L)
        comb = comb_v[j, sl]
        if src_ref is not None:
            src_ref[sl] = jnp.bitwise_and(comb, _MASK)
        if dst_ref is not None:
            dst_ref[sl] = jnp.right_shift(comb, _SHIFT)


def _copy_out(accum, out_hbm, cid, roff, sid):
    @pl.when(sid < NS - 1)
    def _full():
        pltpu.sync_copy(accum.at[pl.ds(roff, RPT)],
                        out_hbm.at[cid, pl.ds(roff, RPT)])

    @pl.when(sid == NS - 1)
    def _last():
        pltpu.sync_copy(accum.at[pl.ds(roff, LAST_RPT)],
                        out_hbm.at[cid, pl.ds(roff, LAST_RPT)])


@functools.partial(
    pl.kernel,
    out_type=jax.ShapeDtypeStruct((NC, N, C), jnp.float32),
    mesh=_mesh,
    compiler_params=pltpu.CompilerParams(use_tc_tiling_on_sc=False,
                                         needs_layout_passes=False),
    scratch_types=[
        pltpu.VMEM((NBMAX2, KB2), jnp.int32),
        pltpu.VMEM((2, KB2), jnp.int32),
        pltpu.VMEM((2, KB2), jnp.int32),
        pltpu.VMEM((2, KB2, C), jnp.float32),
        pltpu.VMEM_SHARED((NPAD, C), jnp.float32),
        pltpu.SemaphoreType.DMA,
        pltpu.SemaphoreType.DMA,
    ],
)
def _sc2_segsum(y_hbm, comb_hbm, zeros_hbm, out_hbm,
                comb_v, srcb, dstb, rows_v, accum, sem0, sem1):
    cid = lax.axis_index("c")
    sid = lax.axis_index("s")
    roff = pl.multiple_of(sid * RPT, 8)
    sems = (sem0, sem1)
    nb = jnp.where(cid == 0, NB2_0, NB2_1)

    pltpu.sync_copy(zeros_hbm.at[pl.ds(roff, RPT)], accum.at[pl.ds(roff, RPT)])

    @pl.when(cid == 0)
    def _stage0():
        pltpu.sync_copy(comb_hbm.at[sid, pl.ds(0, NB2_0)],
                        comb_v.at[pl.ds(0, NB2_0)])

    @pl.when(cid == 1)
    def _stage1():
        pltpu.sync_copy(comb_hbm.at[sid, pl.ds(NB2_0, NB2_1)],
                        comb_v.at[pl.ds(0, NB2_1)])

    plsc.subcore_barrier()

    for b in range(2):
        _decode_batch(comb_v, b, srcb.at[b], dstb.at[b], KB2)
        pltpu.async_copy(y_hbm.at[srcb.at[b]], rows_v.at[b], sems[b])

    def _step(j, b):
        pltpu.make_async_copy(y_hbm.at[srcb.at[b]], rows_v.at[b],
                              sems[b]).wait()
        pltpu.sync_copy(rows_v.at[b], accum.at[dstb.at[b]], add=True)

        @pl.when(j + 2 < nb)
        def _prefetch():
            _decode_batch(comb_v, j + 2, srcb.at[b], dstb.at[b], KB2)
            pltpu.async_copy(y_hbm.at[srcb.at[b]], rows_v.at[b], sems[b])

    def body(jj, _):
        for b in range(2):
            _step(jj * 2 + b, b)
        return _

    lax.fori_loop(0, nb // 2, body, None)
    plsc.subcore_barrier()
    _copy_out(accum, out_hbm, cid, roff, sid)


@functools.partial(
    pl.kernel,
    out_type=jax.ShapeDtypeStruct((NC, N, C), jnp.float32),
    mesh=_mesh,
    compiler_params=pltpu.CompilerParams(use_tc_tiling_on_sc=False,
                                         needs_layout_passes=False),
    scratch_types=[
        pltpu.VMEM((NBMAX4, KB), jnp.int32),
        pltpu.VMEM((2, KB), jnp.int32),
        pltpu.VMEM((2, KB), jnp.int32),
        pltpu.VMEM((2, KB, H), jnp.float32),
        pltpu.VMEM((2, KB, H), jnp.float32),
        pltpu.VMEM((2, KB, C), jnp.float32),
        pltpu.VMEM((H,), jnp.float32),
        pltpu.VMEM((L,), jnp.float32),
        pltpu.VMEM_SHARED((NPAD, C), jnp.float32),
        pltpu.SemaphoreType.DMA,
        pltpu.SemaphoreType.DMA,
        pltpu.SemaphoreType.DMA,
        pltpu.SemaphoreType.DMA,
        pltpu.SemaphoreType.DMA,
        pltpu.SemaphoreType.DMA,
    ],
)
def _sc4_edge_mlp(y_hbm, u_hbm, v_hbm, comb_hbm, w2b_hbm, b2b_hbm,
                  zeros_hbm, out_hbm,
                  comb_v, srcb, dstb, u_v, v_v, y_v, w2_v, b2_v, accum,
                  sem_u0, sem_u1, sem_v0, sem_v1, sem_y0, sem_y1):
    cid = lax.axis_index("c")
    sid = lax.axis_index("s")
    roff = pl.multiple_of(sid * RPT, 8)
    sems_u = (sem_u0, sem_u1)
    sems_v = (sem_v0, sem_v1)
    sems_y = (sem_y0, sem_y1)
    nb = jnp.where(cid == 0, NB4_0, NB4_1)

    pltpu.sync_copy(zeros_hbm.at[pl.ds(roff, RPT)], accum.at[pl.ds(roff, RPT)])

    @pl.when(cid == 0)
    def _stage0():
        pltpu.sync_copy(comb_hbm.at[sid, pl.ds(0, NB4_0)],
                        comb_v.at[pl.ds(0, NB4_0)])

    @pl.when(cid == 1)
    def _stage1():
        pltpu.sync_copy(comb_hbm.at[sid, pl.ds(NB4_0, NB4_1)],
                        comb_v.at[pl.ds(0, NB4_1)])

    pltpu.sync_copy(w2b_hbm, w2_v)
    pltpu.sync_copy(b2b_hbm, b2_v)
    plsc.subcore_barrier()

    b2vec = b2_v[...]
    NKC = H // L
    w2cs = [w2_v[pl.ds(c * L, L)] for c in range(NKC)]
    EU = 8

    def _issue(b):
        pltpu.async_copy(u_hbm.at[srcb.at[b]], u_v.at[b], sems_u[b])
        pltpu.async_copy(v_hbm.at[dstb.at[b]], v_v.at[b], sems_v[b])
        pltpu.async_copy(y_hbm.at[srcb.at[b]], y_v.at[b], sems_y[b])

    for b in range(2):
        _decode_batch(comb_v, b, srcb.at[b], dstb.at[b])
        _issue(b)

    def body(jj, _):
        for b in range(2):
            j = jj * 2 + b
            pltpu.make_async_copy(u_hbm.at[srcb.at[b]], u_v.at[b],
                                  sems_u[b]).wait()
            pltpu.make_async_copy(v_hbm.at[dstb.at[b]], v_v.at[b],
                                  sems_v[b]).wait()
            pltpu.make_async_copy(y_hbm.at[srcb.at[b]], y_v.at[b],
                                  sems_y[b]).wait()

            def ebody(eb, _):
                for t in range(EU):
                    e = eb * EU + t
                    acc0 = b2vec
                    acc1 = jnp.zeros((L,), jnp.float32)
                    for c in range(NKC):
                        u_c = u_v[b, e, pl.ds(c * L, L)]
                        v_c = v_v[b, e, pl.ds(c * L, L)]
                        g_c = jnp.maximum(u_c + v_c, 0.0) * w2cs[c]
                        if c % 2 == 0:
                            acc0 = acc0 + g_c
                        else:
                            acc1 = acc1 + g_c
                    ew = jnp.maximum(jnp.sum(acc0 + acc1), 0.0)
                    for cc in range(C // L):
                        sl = pl.ds(cc * L, L)
                        y_v[b, e, sl] = y_v[b, e, sl] * ew
                return _

            lax.fori_loop(0, KB // EU, ebody, None)
            pltpu.sync_copy(y_v.at[b], accum.at[dstb.at[b]], add=True)

            @pl.when(j + 2 < nb)
            def _prefetch():
                _decode_batch(comb_v, j + 2, srcb.at[b], dstb.at[b])
                _issue(b)
        return _

    lax.fori_loop(0, nb // 2, body, None)
    plsc.subcore_barrier()
    _copy_out(accum, out_hbm, cid, roff, sid)


def kernel(x, edge_index, W_gcn, b_gcn, W1, b1, W2, b2):
    src = edge_index[0]
    dst = edge_index[1]
    pad = EPAD - E
    src_p = jnp.concatenate([src, jnp.zeros((pad,), jnp.int32)])
    dst_p = jnp.concatenate([dst, jnp.full((pad,), NPAD - 1, jnp.int32)])
    comb_flat = dst_p * (1 << _SHIFT) + src_p
    comb2 = comb_flat.reshape(NS, NBT2, KB2)
    comb4 = comb_flat.reshape(NS, NBT4, KB)
    zeros = jnp.zeros((NPAD, C), jnp.float32)

    w1a = W1[:C]
    w1b = W1[C:]
    bg2 = b_gcn.reshape(1, C)
    b12 = b1.reshape(1, H)
    w2b = W2.reshape(H)
    b2b = jnp.broadcast_to(b2.reshape(1) / L, (L,))

    y = _tc1_y(x, W_gcn)
    part1 = _sc2_segsum(y, comb2, zeros)
    u, v = _tc3_uv(part1, bg2, w1a, w1b, b12)
    part2 = _sc4_edge_mlp(y, u, v, comb4, w2b, b2b, zeros)
    return _tc5_out(part2, bg2)


# speedup vs baseline: 1.1528x; 1.1528x over previous
import functools

import jax
import jax.numpy as jnp
from jax import lax
from jax.experimental import pallas as pl
from jax.experimental.pallas import tpu as pltpu
from jax.experimental.pallas import tpu_sc as plsc

N, E, D, C = 10000, 320000, 128, 64
H = 4 * C

NC, NS, L = 2, 16, 16
NW = NC * NS
NPAD = 10240
RPT = NPAD // NS
KB = 64
EPW = 10112
NBATCH = EPW // KB
KB2 = 128
NB2 = EPW // KB2
LAST_RPT = N - (NS - 1) * RPT

NBT4 = 2 * NBATCH
NB4_0, NB4_1 = 176, 140
NBMAX4 = max(NB4_0, NB4_1)
NBT2 = 2 * NB2
NB2_0, NB2_1 = 102, 56
NBMAX2 = max(NB2_0, NB2_1)
assert NB4_0 + NB4_1 == NBT4 and NB4_0 % 2 == 0 and NB4_1 % 2 == 0
assert NB2_0 + NB2_1 == NBT2 and NB2_0 % 2 == 0 and NB2_1 % 2 == 0
EPAD = EPW * NW

_mesh = plsc.VectorSubcoreMesh(
    core_axis_name="c", subcore_axis_name="s", num_cores=NC, num_subcores=NS)


def _tc1_body(x_ref, w_ref, o_ref):
    o_ref[...] = jnp.dot(x_ref[...], w_ref[...],
                         preferred_element_type=jnp.float32,
                         precision=jax.lax.Precision.HIGHEST)


def _tc1_y(x, w_gcn):
    rb = 1000
    return pl.pallas_call(
        _tc1_body,
        grid=(N // rb,),
        in_specs=[
            pl.BlockSpec((rb, D), lambda i: (i, 0)),
            pl.BlockSpec((D, C), lambda i: (0, 0)),
        ],
        out_specs=pl.BlockSpec((rb, C), lambda i: (i, 0)),
        out_shape=jax.ShapeDtypeStruct((N, C), jnp.float32),
    )(x, w_gcn)


def _tc3_body(p_ref, bg_ref, w1a_ref, w1b_ref, b1_ref, u_ref, v_ref):
    emb = p_ref[0] + p_ref[1] + bg_ref[...]
    u_ref[...] = jnp.dot(emb, w1a_ref[...],
                         preferred_element_type=jnp.float32,
                         precision=jax.lax.Precision.HIGHEST) + b1_ref[...]
    v_ref[...] = jnp.dot(emb, w1b_ref[...],
                         preferred_element_type=jnp.float32,
                         precision=jax.lax.Precision.HIGHEST)


def _tc3_uv(partial, b_gcn, w1a, w1b, b1):
    rb = 1000
    return pl.pallas_call(
        _tc3_body,
        grid=(N // rb,),
        in_specs=[
            pl.BlockSpec((NC, rb, C), lambda i: (0, i, 0)),
            pl.BlockSpec((1, C), lambda i: (0, 0)),
            pl.BlockSpec((C, H), lambda i: (0, 0)),
            pl.BlockSpec((C, H), lambda i: (0, 0)),
            pl.BlockSpec((1, H), lambda i: (0, 0)),
        ],
        out_specs=[
            pl.BlockSpec((rb, H), lambda i: (i, 0)),
            pl.BlockSpec((rb, H), lambda i: (i, 0)),
        ],
        out_shape=[
            jax.ShapeDtypeStruct((N, H), jnp.float32),
            jax.ShapeDtypeStruct((N, H), jnp.float32),
        ],
    )(partial, b_gcn, w1a, w1b, b1)


def _tc5_body(q_ref, bg_ref, o_ref):
    o_ref[...] = q_ref[0] + q_ref[1] + bg_ref[...]


def _tc5_out(partial, b_gcn):
    rb = 1000
    return pl.pallas_call(
        _tc5_body,
        grid=(N // rb,),
        in_specs=[
            pl.BlockSpec((NC, rb, C), lambda i: (0, i, 0)),
            pl.BlockSpec((1, C), lambda i: (0, 0)),
        ],
        out_specs=pl.BlockSpec((rb, C), lambda i: (i, 0)),
        out_shape=jax.ShapeDtypeStruct((N, C), jnp.float32),
    )(partial, b_gcn)


_SHIFT = 14
_MASK = (1 << _SHIFT) - 1


def _decode_batch(comb_v, j, src_ref, dst_ref, kb=KB):
    for c in range(kb // L):
        sl = pl.ds(c * L, L)
        comb = comb_v[j, sl]
        if src_ref is not None:
            src_ref[sl] = jnp.bitwise_and(comb, _MASK)
        if dst_ref is not None:
            dst_ref[sl] = jnp.right_shift(comb, _SHIFT)


def _copy_out(accum, out_hbm, cid, roff, sid):
    @pl.when(sid < NS - 1)
    def _full():
        pltpu.sync_copy(accum.at[pl.ds(roff, RPT)],
                        out_hbm.at[cid, pl.ds(roff, RPT)])

    @pl.when(sid == NS - 1)
    def _last():
        pltpu.sync_copy(accum.at[pl.ds(roff, LAST_RPT)],
                        out_hbm.at[cid, pl.ds(roff, LAST_RPT)])


@functools.partial(
    pl.kernel,
    out_type=jax.ShapeDtypeStruct((NC, N, C), jnp.float32),
    mesh=_mesh,
    compiler_params=pltpu.CompilerParams(use_tc_tiling_on_sc=False,
                                         needs_layout_passes=False),
    scratch_types=[
        pltpu.VMEM((NBMAX2, KB2), jnp.int32),
        pltpu.VMEM((2, KB2), jnp.int32),
        pltpu.VMEM((2, KB2), jnp.int32),
        pltpu.VMEM((2, KB2, C), jnp.float32),
        pltpu.VMEM_SHARED((NPAD, C), jnp.float32),
        pltpu.SemaphoreType.DMA,
        pltpu.SemaphoreType.DMA,
    ],
)
def _sc2_segsum(y_hbm, comb_hbm, zeros_hbm, out_hbm,
                comb_v, srcb, dstb, rows_v, accum, sem0, sem1):
    cid = lax.axis_index("c")
    sid = lax.axis_index("s")
    roff = pl.multiple_of(sid * RPT, 8)
    sems = (sem0, sem1)
    nb = jnp.where(cid == 0, NB2_0, NB2_1)

    pltpu.sync_copy(zeros_hbm.at[pl.ds(roff, RPT)], accum.at[pl.ds(roff, RPT)])

    @pl.when(cid == 0)
    def _stage0():
        pltpu.sync_copy(comb_hbm.at[sid, pl.ds(0, NB2_0)],
                        comb_v.at[pl.ds(0, NB2_0)])

    @pl.when(cid == 1)
    def _stage1():
        pltpu.sync_copy(comb_hbm.at[sid, pl.ds(NB2_0, NB2_1)],
                        comb_v.at[pl.ds(0, NB2_1)])

    plsc.subcore_barrier()

    for b in range(2):
        _decode_batch(comb_v, b, srcb.at[b], dstb.at[b], KB2)
        pltpu.async_copy(y_hbm.at[srcb.at[b]], rows_v.at[b], sems[b])

    def _step(j, b):
        pltpu.make_async_copy(y_hbm.at[srcb.at[b]], rows_v.at[b],
                              sems[b]).wait()
        pltpu.sync_copy(rows_v.at[b], accum.at[dstb.at[b]], add=True)

        @pl.when(j + 2 < nb)
        def _prefetch():
            _decode_batch(comb_v, j + 2, srcb.at[b], dstb.at[b], KB2)
            pltpu.async_copy(y_hbm.at[srcb.at[b]], rows_v.at[b], sems[b])

    def body(jj, _):
        for b in range(2):
            _step(jj * 2 + b, b)
        return _

    lax.fori_loop(0, nb // 2, body, None)
    plsc.subcore_barrier()
    _copy_out(accum, out_hbm, cid, roff, sid)


@functools.partial(
    pl.kernel,
    out_type=jax.ShapeDtypeStruct((NC, N, C), jnp.float32),
    mesh=_mesh,
    compiler_params=pltpu.CompilerParams(use_tc_tiling_on_sc=False,
                                         needs_layout_passes=False),
    scratch_types=[
        pltpu.VMEM((NBMAX4, KB), jnp.int32),
        pltpu.VMEM((2, KB), jnp.int32),
        pltpu.VMEM((2, KB), jnp.int32),
        pltpu.VMEM((2, KB, H), jnp.float32),
        pltpu.VMEM((2, KB, H), jnp.float32),
        pltpu.VMEM((2, KB, C), jnp.float32),
        pltpu.VMEM((H,), jnp.float32),
        pltpu.VMEM((L,), jnp.float32),
        pltpu.VMEM_SHARED((NPAD, C), jnp.float32),
        pltpu.SemaphoreType.DMA,
        pltpu.SemaphoreType.DMA,
        pltpu.SemaphoreType.DMA,
        pltpu.SemaphoreType.DMA,
        pltpu.SemaphoreType.DMA,
        pltpu.SemaphoreType.DMA,
    ],
)
def _sc4_edge_mlp(y_hbm, u_hbm, v_hbm, comb_hbm, w2b_hbm, b2b_hbm,
                  zeros_hbm, out_hbm,
                  comb_v, srcb, dstb, u_v, v_v, y_v, w2_v, b2_v, accum,
                  sem_u0, sem_u1, sem_v0, sem_v1, sem_y0, sem_y1):
    cid = lax.axis_index("c")
    sid = lax.axis_index("s")
    roff = pl.multiple_of(sid * RPT, 8)
    sems_u = (sem_u0, sem_u1)
    sems_v = (sem_v0, sem_v1)
    sems_y = (sem_y0, sem_y1)
    nb = jnp.where(cid == 0, NB4_0, NB4_1)

    pltpu.sync_copy(zeros_hbm.at[pl.ds(roff, RPT)], accum.at[pl.ds(roff, RPT)])

    @pl.when(cid == 0)
    def _stage0():
        pltpu.sync_copy(comb_hbm.at[sid, pl.ds(0, NB4_0)],
                        comb_v.at[pl.ds(0, NB4_0)])

    @pl.when(cid == 1)
    def _stage1():
        pltpu.sync_copy(comb_hbm.at[sid, pl.ds(NB4_0, NB4_1)],
                        comb_v.at[pl.ds(0, NB4_1)])

    pltpu.sync_copy(w2b_hbm, w2_v)
    pltpu.sync_copy(b2b_hbm, b2_v)
    plsc.subcore_barrier()

    b2vec = b2_v[...]
    NKC = H // L
    w2cs = [w2_v[pl.ds(c * L, L)] for c in range(NKC)]
    EU = 8

    def _issue(b):
        pltpu.async_copy(u_hbm.at[srcb.at[b]], u_v.at[b], sems_u[b])
        pltpu.async_copy(v_hbm.at[dstb.at[b]], v_v.at[b], sems_v[b])
        pltpu.async_copy(y_hbm.at[srcb.at[b]], y_v.at[b], sems_y[b])

    for b in range(2):
        _decode_batch(comb_v, b, srcb.at[b], dstb.at[b])
        _issue(b)

    def body(jj, _):
        for b in range(2):
            j = jj * 2 + b
            pltpu.make_async_copy(u_hbm.at[srcb.at[b]], u_v.at[b],
                                  sems_u[b]).wait()
            pltpu.make_async_copy(v_hbm.at[dstb.at[b]], v_v.at[b],
                                  sems_v[b]).wait()
            pltpu.make_async_copy(y_hbm.at[srcb.at[b]], y_v.at[b],
                                  sems_y[b]).wait()

            def ebody(eb, _):
                for t in range(EU):
                    e = eb * EU + t
                    acc0 = b2vec
                    acc1 = jnp.zeros((L,), jnp.float32)
                    for c in range(NKC):
                        u_c = u_v[b, e, pl.ds(c * L, L)]
                        v_c = v_v[b, e, pl.ds(c * L, L)]
                        g_c = jnp.maximum(u_c + v_c, 0.0) * w2cs[c]
                        if c % 2 == 0:
                            acc0 = acc0 + g_c
                        else:
                            acc1 = acc1 + g_c
                    ew = jnp.maximum(jnp.sum(acc0 + acc1), 0.0)
                    for cc in range(C // L):
                        sl = pl.ds(cc * L, L)
                        y_v[b, e, sl] = y_v[b, e, sl] * ew
                return _

            lax.fori_loop(0, KB // EU, ebody, None)
            pltpu.sync_copy(y_v.at[b], accum.at[dstb.at[b]], add=True)

            @pl.when(j + 2 < nb)
            def _prefetch():
                _decode_batch(comb_v, j + 2, srcb.at[b], dstb.at[b])
                _issue(b)
        return _

    lax.fori_loop(0, nb // 2, body, None)
    plsc.subcore_barrier()
    _copy_out(accum, out_hbm, cid, roff, sid)


def kernel(x, edge_index, W_gcn, b_gcn, W1, b1, W2, b2):
    src = edge_index[0]
    dst = edge_index[1]
    pad = EPAD - E
    src_p = jnp.concatenate([src, jnp.zeros((pad,), jnp.int32)])
    dst_p = jnp.concatenate([dst, jnp.full((pad,), NPAD - 1, jnp.int32)])
    comb_flat = dst_p * (1 << _SHIFT) + src_p
    comb2 = comb_flat.reshape(NS, NBT2, KB2)
    comb4 = comb_flat.reshape(NS, NBT4, KB)
    zeros = jnp.zeros((NPAD, C), jnp.float32)

    w1a = W1[:C]
    w1b = W1[C:]
    bg2 = b_gcn.reshape(1, C)
    b12 = b1.reshape(1, H)
    w2b = W2.reshape(H)
    b2b = jnp.broadcast_to(b2.reshape(1) / L, (L,))

    y = _tc1_y(x, W_gcn)
    part1 = _sc2_segsum(y, comb2, zeros)
    u, v = _tc3_uv(part1, bg2, w1a, w1b, b12)
    part2 = _sc4_edge_mlp(y, u, v, comb4, w2b, b2b, zeros)
    return _tc5_out(part2, bg2)
